# baseline (speedup 1.0000x reference)
import functools

import jax
import jax.numpy as jnp
from jax import lax
from jax.experimental import pallas as pl
from jax.experimental.pallas import tpu as pltpu
from jax.experimental.pallas import tpu_sc as plsc

_B = 16384
_D = 32
_V = 1000000
_CW = 512
_NCHUNK_FULL = _V // _CW
_TAIL_Q = _NCHUNK_FULL
_TAIL_COL = _TAIL_Q * _CW
_CAP = 1024


@functools.cache
def _build():
    info = plsc.get_sparse_core_info()
    nw = info.num_cores * info.num_subcores
    assert nw == 32
    b_per_w = _B // nw
    mesh = plsc.VectorSubcoreMesh(core_axis_name="c", subcore_axis_name="s")

    @functools.partial(
        pl.kernel,
        mesh=mesh,
        compiler_params=pltpu.CompilerParams(needs_layout_passes=False),
        out_type=(
            jax.ShapeDtypeStruct((nw * _CAP // 4, 128), jnp.float32),
            jax.ShapeDtypeStruct((nw * _CAP,), jnp.int32),
        ),
        scratch_types=[
            pltpu.VMEM((_B,), jnp.int32),
            pltpu.VMEM((_CAP,), jnp.int32),
            pltpu.VMEM((_CAP,), jnp.int32),
            pltpu.VMEM((2, _D, _CW), jnp.float32),
            pltpu.VMEM((_CAP // 4, 128), jnp.float32),
            pltpu.SemaphoreType.DMA,
            pltpu.SemaphoreType.DMA,
            pltpu.SemaphoreType.DMA,
        ],
    )
    def collect_kernel(
        idx_hbm, tab_hbm, vals_hbm, js_hbm,
        idx_v, wl_c, wl_j, blk_v, pv_v, sem_i, sem_a, sem_b,
    ):
        wid = lax.axis_index("s") * info.num_cores + lax.axis_index("c")
        lane = lax.iota(jnp.int32, 16)
        pltpu.async_copy(idx_hbm, idx_v, sem_i).wait()

        neg1 = jnp.full((16,), -1, dtype=jnp.int32)

        @pl.loop(0, _CAP // 16)
        def _init(g):
            wl_j[pl.ds(g * 16, 16)] = neg1

        @pl.loop(0, _B // 16, init_carry=jnp.int32(0))
        def cnt(g, cnt):
            cv = idx_v[pl.ds(g * 16, 16)]
            mine = ((cv >> 9) & (nw - 1)) == wid
            jv = g * 16 + lane
            plsc.store_compressed(wl_c.at[pl.ds(cnt, 16)], cv, mask=mine)
            plsc.store_compressed(wl_j.at[pl.ds(cnt, 16)], jv, mask=mine)
            return cnt + jnp.sum(mine.astype(jnp.int32))

        n_groups = (cnt + 15) >> 4
        n_k = (_TAIL_Q - wid) // nw + 1

        def fire(k, buf, sem):
            q = k * nw + wid
            col = pl.multiple_of(q * _CW, 128)

            @pl.when(q < _TAIL_Q)
            def _full():
                pltpu.async_copy(tab_hbm.at[:, pl.ds(col, _CW)], blk_v.at[buf], sem)

            @pl.when(q == _TAIL_Q)
            def _tail():
                pltpu.async_copy(
                    tab_hbm.at[:, pl.ds(col, 128)],
                    blk_v.at[buf, :, pl.ds(0, 128)],
                    sem,
                )

        def drain(k, sem):
            q = k * nw + wid

            @pl.when(q < _TAIL_Q)
            def _full():
                pltpu.make_async_copy(
                    tab_hbm.at[:, pl.ds(0, _CW)], blk_v.at[0], sem
                ).wait()

            @pl.when(q == _TAIL_Q)
            def _tail():
                pltpu.make_async_copy(
                    tab_hbm.at[:, pl.ds(0, 128)],
                    blk_v.at[0, :, pl.ds(0, 128)],
                    sem,
                ).wait()

        def scan(k, buf):
            q = k * nw + wid
            qlo = q * _CW
            bcol = jnp.full((16,), buf, dtype=jnp.int32)

            @pl.loop(0, n_groups)
            def _grp(g):
                wc = wl_c[pl.ds(g * 16, 16)]
                valid = (g * 16 + lane) < cnt
                mc = ((wc >> 9) == q) & valid
                any_m = jnp.sum(mc.astype(jnp.int32))

                @pl.when(any_m > 0)
                def _hit():
                    mci = mc.astype(jnp.int32)
                    for kk in range(16):
                        sel = lane == kk
                        hit = jnp.sum(jnp.where(sel, mci, 0))

                        @pl.when(hit > 0)
                        def _one():
                            c = jnp.sum(jnp.where(sel, wc, 0))
                            rel = c - qlo
                            rcol = jnp.full((16,), rel, dtype=jnp.int32)
                            lo = plsc.load_gather(blk_v, [bcol, lane, rcol])
                            hi = plsc.load_gather(blk_v, [bcol, lane + 16, rcol])
                            slot = g * 16 + kk
                            w0 = slot * _D
                            wlo = w0 + lane
                            whi = w0 + 16 + lane
                            plsc.store_scatter(
                                pv_v, [wlo >> 7, wlo & 127], lo
                            )
                            plsc.store_scatter(
                                pv_v, [whi >> 7, whi & 127], hi
                            )

        fire(0, 0, sem_a)

        @pl.loop(0, n_k)
        def _chunk(k):
            @pl.when((k & 1) == 0)
            def _even():
                @pl.when(k + 1 < n_k)
                def _pf():
                    fire(k + 1, 1, sem_b)

                drain(k, sem_a)
                scan(k, 0)

            @pl.when((k & 1) == 1)
            def _odd():
                @pl.when(k + 1 < n_k)
                def _pf():
                    fire(k + 1, 0, sem_a)

                drain(k, sem_b)
                scan(k, 1)

        vbase = pl.multiple_of(wid * (_CAP // 4), 128)
        jbase = pl.multiple_of(wid * _CAP, 128)
        pltpu.sync_copy(pv_v, vals_hbm.at[pl.ds(vbase, _CAP // 4)])
        pltpu.sync_copy(wl_j, js_hbm.at[pl.ds(jbase, _CAP)])

    @functools.partial(
        pl.kernel,
        mesh=mesh,
        compiler_params=pltpu.CompilerParams(
            use_tc_tiling_on_sc=False, needs_layout_passes=False
        ),
        out_type=jax.ShapeDtypeStruct((_B, _D), jnp.float32),
        scratch_types=[
            pltpu.VMEM((nw * _CAP,), jnp.int32),
            pltpu.VMEM((b_per_w,), jnp.int32),
            pltpu.VMEM((b_per_w,), jnp.int32),
            pltpu.VMEM((b_per_w, _D), jnp.float32),
            pltpu.VMEM((b_per_w, _D), jnp.float32),
            pltpu.SemaphoreType.DMA,
            pltpu.SemaphoreType.DMA,
        ],
    )
    def distribute_kernel(
        js_hbm, vals_hbm, out_hbm,
        pj_v, rid_v, jl_v, got_v, out_v, sem_i, sem,
    ):
        wid = lax.axis_index("s") * info.num_cores + lax.axis_index("c")
        lane = lax.iota(jnp.int32, 16)
        lo_j = wid * b_per_w
        pltpu.async_copy(js_hbm, pj_v, sem_i).wait()

        @pl.loop(0, (nw * _CAP) // 16, init_carry=jnp.int32(0))
        def cnt(g, cnt):
            pj = pj_v[pl.ds(g * 16, 16)]
            mine = (pj >= lo_j) & (pj < lo_j + b_per_w)
            rows = g * 16 + lane
            plsc.store_compressed(rid_v.at[pl.ds(cnt, 16)], rows, mask=mine)
            plsc.store_compressed(jl_v.at[pl.ds(cnt, 16)], pj, mask=mine)
            return cnt + jnp.sum(mine.astype(jnp.int32))

        pltpu.async_copy(vals_hbm.at[rid_v], got_v, sem).wait()

        @pl.loop(0, b_per_w // 16)
        def _place(g):
            jv = jl_v[pl.ds(g * 16, 16)]
            for kk in range(16):
                sel = lane == kk
                i2 = g * 16 + kk
                jloc = jnp.sum(jnp.where(sel, jv, 0)) - lo_j
                icol = jnp.full((16,), i2, dtype=jnp.int32)
                jcol = jnp.full((16,), jloc, dtype=jnp.int32)
                lo = plsc.load_gather(got_v, [icol, lane])
                hi = plsc.load_gather(got_v, [icol, lane + 16])
                plsc.store_scatter(out_v, [jcol, lane], lo)
                plsc.store_scatter(out_v, [jcol, lane + 16], hi)

        pltpu.sync_copy(out_v, out_hbm.at[pl.ds(lo_j, b_per_w)])

    return collect_kernel, distribute_kernel


def kernel(beta, Emb):
    collect, distribute = _build()
    nw = 32
    vals_p, js_p = collect(beta.astype(jnp.int32), Emb.T)
    out = distribute(js_p, vals_p.reshape(nw * _CAP, _D))
    return out


# speedup vs baseline: 2.9851x; 2.9851x over previous
import functools

import jax
import jax.numpy as jnp
from jax import lax
from jax.experimental import pallas as pl
from jax.experimental.pallas import tpu as pltpu
from jax.experimental.pallas import tpu_sc as plsc

_B = 16384
_D = 32
_CHUNK = 8


@functools.cache
def _build():
    info = plsc.get_sparse_core_info()
    nw = info.num_cores * info.num_subcores
    b_per_w = _B // nw
    n_pairs = b_per_w // (2 * _CHUNK)
    mesh = plsc.VectorSubcoreMesh(core_axis_name="c", subcore_axis_name="s")

    @functools.partial(
        pl.kernel,
        mesh=mesh,
        compiler_params=pltpu.CompilerParams(needs_layout_passes=False),
        out_type=jax.ShapeDtypeStruct((_D, _B), jnp.float32),
        scratch_types=[
            pltpu.VMEM((b_per_w,), jnp.int32),
            pltpu.VMEM((2, _CHUNK, _D, 128), jnp.float32),
            pltpu.VMEM((_D, b_per_w), jnp.float32),
            pltpu.SemaphoreType.DMA,
            pltpu.SemaphoreType.DMA,
            pltpu.SemaphoreType.DMA,
        ],
    )
    def gather_kernel(
        idx_hbm, tab_hbm, out_hbm, idx_v, blk_v, rows_v, sem_i, sem_a, sem_b
    ):
        wid = lax.axis_index("s") * info.num_cores + lax.axis_index("c")
        base = pl.multiple_of(wid * b_per_w, 128)
        pltpu.async_copy(idx_hbm.at[pl.ds(base, b_per_w)], idx_v, sem_i).wait()
        lane = lax.iota(jnp.int32, 16)

        def fire(cv, half, buf, sem):
            for kk in range(_CHUNK):
                c = jnp.sum(jnp.where(lane == half * _CHUNK + kk, cv, 0))
                t = pl.multiple_of(c - c % 128, 128)
                pltpu.async_copy(
                    tab_hbm.at[:, pl.ds(t, 128)], blk_v.at[buf, kk], sem
                )

        def drain(sem):
            for kk in range(_CHUNK):
                pltpu.make_async_copy(
                    tab_hbm.at[:, pl.ds(0, 128)], blk_v.at[0, kk], sem
                ).wait()

        def extract(cv, half, buf, jbase):
            for kk in range(_CHUNK):
                c = jnp.sum(jnp.where(lane == half * _CHUNK + kk, cv, 0))
                r = c % 128
                kcol = jnp.full((16,), kk, dtype=jnp.int32)
                bcol = jnp.full((16,), buf, dtype=jnp.int32)
                rcol = jnp.full((16,), r, dtype=jnp.int32)
                jcol = jnp.full((16,), jbase + kk, dtype=jnp.int32)
                lo = plsc.load_gather(blk_v, [bcol, kcol, lane, rcol])
                hi = plsc.load_gather(blk_v, [bcol, kcol, lane + 16, rcol])
                plsc.store_scatter(rows_v, [lane, jcol], lo)
                plsc.store_scatter(rows_v, [lane + 16, jcol], hi)

        cv0 = idx_v[pl.ds(0, 16)]
        fire(cv0, 0, 0, sem_a)

        @pl.loop(0, n_pairs)
        def _pair(pp):
            cv = idx_v[pl.ds(pp * 16, 16)]
            jbase = pp * 16
            fire(cv, 1, 1, sem_b)
            drain(sem_a)
            extract(cv, 0, 0, jbase)

            @pl.when(pp < n_pairs - 1)
            def _prefetch():
                cvn = idx_v[pl.ds(pp * 16 + 16, 16)]
                fire(cvn, 0, 0, sem_a)

            drain(sem_b)
            extract(cv, 1, 1, jbase + _CHUNK)

        pltpu.sync_copy(rows_v, out_hbm.at[:, pl.ds(base, b_per_w)])

    return gather_kernel


def kernel(beta, Emb):
    out_t = _build()(beta.astype(jnp.int32), Emb.T)
    return out_t.T


# speedup vs baseline: 3.2450x; 1.0871x over previous
import functools

import jax
import jax.numpy as jnp
from jax import lax
from jax.experimental import pallas as pl
from jax.experimental.pallas import tpu as pltpu
from jax.experimental.pallas import tpu_sc as plsc

_B = 16384
_D = 32
_CHUNK = 8


@functools.cache
def _build():
    info = plsc.get_sparse_core_info()
    nw = info.num_cores * info.num_subcores
    b_per_w = _B // nw
    n_pairs = b_per_w // (2 * _CHUNK)
    mesh = plsc.VectorSubcoreMesh(core_axis_name="c", subcore_axis_name="s")

    @functools.partial(
        pl.kernel,
        mesh=mesh,
        compiler_params=pltpu.CompilerParams(needs_layout_passes=False),
        out_type=jax.ShapeDtypeStruct((_D, _B), jnp.float32),
        scratch_types=[
            pltpu.VMEM((b_per_w,), jnp.int32),
            pltpu.VMEM((3, _CHUNK, _D, 128), jnp.float32),
            pltpu.VMEM((_D, b_per_w), jnp.float32),
            pltpu.SemaphoreType.DMA,
            pltpu.SemaphoreType.DMA,
            pltpu.SemaphoreType.DMA,
            pltpu.SemaphoreType.DMA,
        ],
    )
    def gather_kernel(
        idx_hbm, tab_hbm, out_hbm, idx_v, blk_v, rows_v, sem_i, sem_a, sem_b, sem_c
    ):
        wid = lax.axis_index("s") * info.num_cores + lax.axis_index("c")
        base = pl.multiple_of(wid * b_per_w, 128)
        pltpu.async_copy(idx_hbm.at[pl.ds(base, b_per_w)], idx_v, sem_i).wait()
        lane = lax.iota(jnp.int32, 16)
        n_chunks = b_per_w // _CHUNK
        sems = (sem_a, sem_b, sem_c)

        def fire(k, buf, sem):
            cv = idx_v[pl.ds((k // 2) * 16, 16)]
            half = k & 1
            for kk in range(_CHUNK):
                c = jnp.sum(jnp.where(lane == half * _CHUNK + kk, cv, 0))
                t = pl.multiple_of(c - c % 128, 128)
                pltpu.async_copy(
                    tab_hbm.at[:, pl.ds(t, 128)], blk_v.at[buf, kk], sem
                )

        def drain(sem):
            for kk in range(_CHUNK):
                pltpu.make_async_copy(
                    tab_hbm.at[:, pl.ds(0, 128)], blk_v.at[0, kk], sem
                ).wait()

        def extract(k, buf):
            cv = idx_v[pl.ds((k // 2) * 16, 16)]
            half = k & 1
            jbase = k * _CHUNK
            for kk in range(_CHUNK):
                c = jnp.sum(jnp.where(lane == half * _CHUNK + kk, cv, 0))
                r = c % 128
                kcol = jnp.full((16,), kk, dtype=jnp.int32)
                bcol = jnp.full((16,), buf, dtype=jnp.int32)
                rcol = jnp.full((16,), r, dtype=jnp.int32)
                jcol = jnp.full((16,), jbase + kk, dtype=jnp.int32)
                lo = plsc.load_gather(blk_v, [bcol, kcol, lane, rcol])
                hi = plsc.load_gather(blk_v, [bcol, kcol, lane + 16, rcol])
                plsc.store_scatter(rows_v, [lane, jcol], lo)
                plsc.store_scatter(rows_v, [lane + 16, jcol], hi)

        fire(0, 0, sem_a)
        fire(1, 1, sem_b)

        @pl.loop(0, n_chunks)
        def _chunk(k):
            for s in range(3):
                @pl.when((k % 3) == s)
                def _slot(s=s):
                    @pl.when(k + 2 < n_chunks)
                    def _pf():
                        fire(k + 2, (s + 2) % 3, sems[(s + 2) % 3])

                    drain(sems[s])
                    extract(k, s)

        pltpu.sync_copy(rows_v, out_hbm.at[:, pl.ds(base, b_per_w)])

    return gather_kernel


def kernel(beta, Emb):
    out_t = _build()(beta.astype(jnp.int32), Emb.T)
    return out_t.T


# speedup vs baseline: 3.5585x; 1.0966x over previous
import functools

import jax
import jax.numpy as jnp
from jax import lax
from jax.experimental import pallas as pl
from jax.experimental.pallas import tpu as pltpu
from jax.experimental.pallas import tpu_sc as plsc

_B = 16384
_D = 32
_CHUNK = 4
_SLOTS = 6


@functools.cache
def _build():
    info = plsc.get_sparse_core_info()
    nw = info.num_cores * info.num_subcores
    b_per_w = _B // nw
    n_pairs = b_per_w // (2 * _CHUNK)
    mesh = plsc.VectorSubcoreMesh(core_axis_name="c", subcore_axis_name="s")

    @functools.partial(
        pl.kernel,
        mesh=mesh,
        compiler_params=pltpu.CompilerParams(needs_layout_passes=False),
        out_type=jax.ShapeDtypeStruct((_D, _B), jnp.float32),
        scratch_types=[
            pltpu.VMEM((b_per_w,), jnp.int32),
            pltpu.VMEM((_SLOTS, _CHUNK, _D, 128), jnp.float32),
            pltpu.VMEM((_D, b_per_w), jnp.float32),
            pltpu.SemaphoreType.DMA,
        ] + [pltpu.SemaphoreType.DMA] * _SLOTS,
    )
    def gather_kernel(
        idx_hbm, tab_hbm, out_hbm, idx_v, blk_v, rows_v, sem_i, *sems
    ):
        wid = lax.axis_index("s") * info.num_cores + lax.axis_index("c")
        base = pl.multiple_of(wid * b_per_w, 128)
        pltpu.async_copy(idx_hbm.at[pl.ds(base, b_per_w)], idx_v, sem_i).wait()
        lane = lax.iota(jnp.int32, 16)
        n_chunks = b_per_w // _CHUNK
        per_cv = 16 // _CHUNK

        def fire(k, buf, sem):
            cv = idx_v[pl.ds((k // per_cv) * 16, 16)]
            half = k % per_cv
            for kk in range(_CHUNK):
                c = jnp.sum(jnp.where(lane == half * _CHUNK + kk, cv, 0))
                t = pl.multiple_of(c - c % 128, 128)
                pltpu.async_copy(
                    tab_hbm.at[:, pl.ds(t, 128)], blk_v.at[buf, kk], sem
                )

        def drain(sem):
            for kk in range(_CHUNK):
                pltpu.make_async_copy(
                    tab_hbm.at[:, pl.ds(0, 128)], blk_v.at[0, kk], sem
                ).wait()

        def extract(k, buf):
            cv = idx_v[pl.ds((k // per_cv) * 16, 16)]
            half = k % per_cv
            jbase = k * _CHUNK
            for kk in range(_CHUNK):
                c = jnp.sum(jnp.where(lane == half * _CHUNK + kk, cv, 0))
                r = c % 128
                kcol = jnp.full((16,), kk, dtype=jnp.int32)
                bcol = jnp.full((16,), buf, dtype=jnp.int32)
                rcol = jnp.full((16,), r, dtype=jnp.int32)
                jcol = jnp.full((16,), jbase + kk, dtype=jnp.int32)
                lo = plsc.load_gather(blk_v, [bcol, kcol, lane, rcol])
                hi = plsc.load_gather(blk_v, [bcol, kcol, lane + 16, rcol])
                plsc.store_scatter(rows_v, [lane, jcol], lo)
                plsc.store_scatter(rows_v, [lane + 16, jcol], hi)

        for pre in range(_SLOTS - 1):
            fire(pre, pre, sems[pre])

        @pl.loop(0, n_chunks)
        def _chunk(k):
            for s in range(_SLOTS):
                @pl.when((k % _SLOTS) == s)
                def _slot(s=s):
                    nxt = (s + _SLOTS - 1) % _SLOTS

                    @pl.when(k + _SLOTS - 1 < n_chunks)
                    def _pf():
                        fire(k + _SLOTS - 1, nxt, sems[nxt])

                    drain(sems[s])
                    extract(k, s)

        pltpu.sync_copy(rows_v, out_hbm.at[:, pl.ds(base, b_per_w)])

    return gather_kernel


def kernel(beta, Emb):
    out_t = _build()(beta.astype(jnp.int32), Emb.T)
    return out_t.T
